# Initial kernel scaffold; baseline (speedup 1.0000x reference)
import jax
import jax.numpy as jnp
from jax.experimental import pallas as pl


def kernel(hidden_states, router_w, w1, w2, w3):
    raise NotImplementedError("write your pallas kernel here")


# speedup vs baseline: 2.5623x; 2.5623x over previous
import functools

import jax
import jax.numpy as jnp
from jax.experimental import pallas as pl
from jax.experimental.pallas import tpu as pltpu

HIDDEN = 1024
INTER = 2048
E = 8
TOPK = 2

TM = 256
TMR = 512


def _router_body(x_ref, rw_ref, logits_ref):
    x = x_ref[...]
    rw = rw_ref[...]
    logits_ref[...] = jax.lax.dot_general(
        x, rw, (((1,), (1,)), ((), ())),
        preferred_element_type=jnp.float32)


def _router_logits(x, router_w):
    T = x.shape[0]
    return pl.pallas_call(
        _router_body,
        grid=(T // TMR,),
        in_specs=[
            pl.BlockSpec((TMR, HIDDEN), lambda i: (i, 0)),
            pl.BlockSpec((E, HIDDEN), lambda i: (0, 0)),
        ],
        out_specs=pl.BlockSpec((TMR, E), lambda i: (i, 0)),
        out_shape=jax.ShapeDtypeStruct((T, E), jnp.float32),
    )(x, router_w)


def _gemm_body(tile_expert_ref, xg_ref, w1_ref, w2_ref, w3_ref, gates_ref,
               yg_ref):
    x = xg_ref[...]
    w1 = w1_ref[0]
    w2 = w2_ref[0]
    w3 = w3_ref[0]
    h1 = jnp.dot(x, w1, preferred_element_type=jnp.float32)
    h2 = jnp.dot(x, w2, preferred_element_type=jnp.float32)
    h = jax.nn.silu(h1) * h2
    y = jnp.dot(h, w3, preferred_element_type=jnp.float32)
    g = gates_ref[0, 0, :]
    yg_ref[...] = y * g[:, None]


def _grouped_gemm(xg, w1, w2, w3, gates_tiles, tile_expert, nt):
    PP = xg.shape[0]
    grid_spec = pltpu.PrefetchScalarGridSpec(
        num_scalar_prefetch=1,
        grid=(nt,),
        in_specs=[
            pl.BlockSpec((TM, HIDDEN), lambda i, te: (i, 0)),
            pl.BlockSpec((1, HIDDEN, INTER), lambda i, te: (te[i], 0, 0)),
            pl.BlockSpec((1, HIDDEN, INTER), lambda i, te: (te[i], 0, 0)),
            pl.BlockSpec((1, INTER, HIDDEN), lambda i, te: (te[i], 0, 0)),
            pl.BlockSpec((1, 1, TM), lambda i, te: (i, 0, 0)),
        ],
        out_specs=pl.BlockSpec((TM, HIDDEN), lambda i, te: (i, 0)),
    )
    return pl.pallas_call(
        _gemm_body,
        grid_spec=grid_spec,
        out_shape=jax.ShapeDtypeStruct((PP, HIDDEN), jnp.float32),
    )(tile_expert, xg, w1, w2, w3, gates_tiles)


def kernel(hidden_states, router_w, w1, w2, w3):
    orig_shape = hidden_states.shape
    x = hidden_states.reshape(-1, HIDDEN)
    T = x.shape[0]
    P = T * TOPK
    NT = P // TM + E
    PP = NT * TM

    router_logits = _router_logits(x, router_w)

    probs = jax.nn.softmax(router_logits, axis=-1)
    topw, sel = jax.lax.top_k(probs, TOPK)
    topw = topw / topw.sum(axis=-1, keepdims=True)

    e_flat = sel.reshape(-1)
    onehot = (e_flat[:, None] == jnp.arange(E)[None, :]).astype(jnp.int32)
    counts = onehot.sum(axis=0)
    rank = jnp.take_along_axis(jnp.cumsum(onehot, axis=0) - 1,
                               e_flat[:, None], axis=1)[:, 0]
    tiles_e = (counts + TM - 1) // TM
    cum_tiles = jnp.cumsum(tiles_e)
    astart = TM * (cum_tiles - tiles_e)
    pos = astart[e_flat] + rank
    slot_token = jnp.zeros((PP,), jnp.int32).at[pos].set(
        jnp.arange(P, dtype=jnp.int32) // TOPK)
    gates_slot = jnp.zeros((PP,), jnp.float32).at[pos].set(topw.reshape(-1))
    tile_expert = jnp.clip(
        jnp.searchsorted(cum_tiles, jnp.arange(NT, dtype=jnp.int32),
                         side="right"),
        0, E - 1).astype(jnp.int32)

    xg = jnp.take(x, slot_token, axis=0)

    yg = _grouped_gemm(xg, w1, w2, w3,
                       gates_slot.reshape(NT, 1, TM), tile_expert, NT)

    pos2 = pos.reshape(T, TOPK)
    out = jnp.take(yg, pos2[:, 0], axis=0) + jnp.take(yg, pos2[:, 1], axis=0)
    return out.reshape(orig_shape), router_logits
